# baseline (speedup 1.0000x reference)
import functools

import jax
import jax.numpy as jnp
from jax import lax
from jax.experimental import pallas as pl
from jax.experimental.pallas import tpu as pltpu
from jax.experimental.pallas import tpu_sc as plsc

N_VARIANTXGENES = 1_000_000
B, S = 16384, 50
D = 64
N_ROWS = B * S
CHUNK = 128
NW = 32
NSC = N_ROWS // (2 * CHUNK)
SC_PER_W = NSC // NW
NB = 4


def _sc_gather(idxe, idxo, table):
    mesh = plsc.VectorSubcoreMesh(core_axis_name="c", subcore_axis_name="s")

    @functools.partial(
        pl.kernel,
        out_type=jax.ShapeDtypeStruct((N_ROWS // 2, 2 * D), jnp.float32),
        mesh=mesh,
        scratch_types=[
            pltpu.VMEM((SC_PER_W, CHUNK), jnp.int32),
            pltpu.VMEM((SC_PER_W, CHUNK), jnp.int32),
            pltpu.VMEM((NB, CHUNK, D), jnp.float32),
            pltpu.VMEM((NB, CHUNK, D), jnp.float32),
            pltpu.SemaphoreType.DMA((NB,)),
            pltpu.SemaphoreType.DMA((NB,)),
        ],
        compiler_params=pltpu.CompilerParams(use_tc_tiling_on_sc=False),
    )
    def k(idxe_hbm, idxo_hbm, tbl, out_hbm, ev, ov, gbe, gbo, gsem, ssem):
        wid = lax.axis_index("s") * 2 + lax.axis_index("c")
        pltpu.sync_copy(idxe_hbm.at[pl.ds(wid * SC_PER_W, SC_PER_W)], ev)
        pltpu.sync_copy(idxo_hbm.at[pl.ds(wid * SC_PER_W, SC_PER_W)], ov)

        def fire(b, sc):
            pltpu.async_copy(tbl.at[ev.at[sc]], gbe.at[b], gsem.at[b])
            pltpu.async_copy(tbl.at[ov.at[sc]], gbo.at[b], gsem.at[b])

        for b in range(NB):
            fire(b, b)

        def group(g, carry):
            for b in range(NB):
                sc = g * NB + b
                pltpu.make_async_copy(tbl.at[ev.at[0]], gbe.at[b], gsem.at[b]).wait()
                pltpu.make_async_copy(tbl.at[ev.at[0]], gbo.at[b], gsem.at[b]).wait()
                p0 = (wid * SC_PER_W + sc) * CHUNK
                pltpu.async_copy(
                    gbe.at[b], out_hbm.at[pl.ds(p0, CHUNK), pl.ds(0, D)], ssem.at[b]
                )
                pltpu.async_copy(
                    gbo.at[b], out_hbm.at[pl.ds(p0, CHUNK), pl.ds(D, D)], ssem.at[b]
                )
                nxt = sc + NB

                @pl.when(nxt < SC_PER_W)
                def _():
                    pltpu.make_async_copy(
                        gbe.at[b], out_hbm.at[pl.ds(0, CHUNK), pl.ds(0, D)],
                        ssem.at[b],
                    ).wait()
                    pltpu.make_async_copy(
                        gbo.at[b], out_hbm.at[pl.ds(0, CHUNK), pl.ds(0, D)],
                        ssem.at[b],
                    ).wait()
                    fire(b, nxt)

            return carry

        lax.fori_loop(0, SC_PER_W // NB, group, 0)

        for b in range(NB):
            pltpu.make_async_copy(
                gbe.at[b], out_hbm.at[pl.ds(0, CHUNK), pl.ds(0, D)], ssem.at[b]
            ).wait()
            pltpu.make_async_copy(
                gbo.at[b], out_hbm.at[pl.ds(0, CHUNK), pl.ds(0, D)], ssem.at[b]
            ).wait()

    return k(idxe, idxo, table)


def kernel(variantxgene_ixs, table):
    idx = variantxgene_ixs.reshape(N_ROWS // 2, 2).astype(jnp.int32)
    idxe = idx[:, 0].reshape(NSC, CHUNK)
    idxo = idx[:, 1].reshape(NSC, CHUNK)
    out = _sc_gather(idxe, idxo, table)
    return out.reshape(B, S, D)


# speedup vs baseline: 1.1601x; 1.1601x over previous
import functools

import jax
import jax.numpy as jnp
from jax import lax
from jax.experimental import pallas as pl
from jax.experimental.pallas import tpu as pltpu
from jax.experimental.pallas import tpu_sc as plsc

N_VARIANTXGENES = 1_000_000
B, S = 16384, 50
D = 64
D_PAD = 128
N_ROWS = B * S
CHUNK = 128
NW = 32
CHUNKS_PER_W = N_ROWS // (CHUNK * NW)
NB = 4


def _sc_gather(idx2d, t_pad):
    mesh = plsc.VectorSubcoreMesh(core_axis_name="c", subcore_axis_name="s")

    @functools.partial(
        pl.kernel,
        out_type=jax.ShapeDtypeStruct((N_ROWS, D), jnp.float32),
        mesh=mesh,
        scratch_types=[
            pltpu.VMEM((CHUNKS_PER_W, CHUNK), jnp.int32),
            pltpu.VMEM((NB, CHUNK, D_PAD), jnp.float32),
            pltpu.SemaphoreType.DMA((NB,)),
            pltpu.SemaphoreType.DMA((NB,)),
        ],
        compiler_params=pltpu.CompilerParams(use_tc_tiling_on_sc=False),
    )
    def k(idx_hbm, tbl, out_hbm, idx_v, gbuf, gsem, ssem):
        wid = lax.axis_index("s") * 2 + lax.axis_index("c")
        pltpu.sync_copy(idx_hbm.at[pl.ds(wid * CHUNKS_PER_W, CHUNKS_PER_W)], idx_v)

        def fire(b, j):
            pltpu.async_copy(tbl.at[idx_v.at[j]], gbuf.at[b], gsem.at[b])

        for b in range(NB):
            fire(b, b)

        def group(g, carry):
            for b in range(NB):
                j = g * NB + b
                pltpu.make_async_copy(
                    tbl.at[idx_v.at[0]], gbuf.at[b], gsem.at[b]
                ).wait()
                base = (wid * CHUNKS_PER_W + j) * CHUNK
                pltpu.async_copy(
                    gbuf.at[b].at[:, pl.ds(0, D)],
                    out_hbm.at[pl.ds(base, CHUNK)], ssem.at[b],
                )
                nxt = j + NB

                @pl.when(nxt < CHUNKS_PER_W)
                def _():
                    pltpu.make_async_copy(
                        gbuf.at[b].at[:, pl.ds(0, D)],
                        out_hbm.at[pl.ds(0, CHUNK)], ssem.at[b],
                    ).wait()
                    fire(b, nxt)

            return carry

        lax.fori_loop(0, CHUNKS_PER_W // NB, group, 0)

        for b in range(NB):
            pltpu.make_async_copy(
                gbuf.at[b].at[:, pl.ds(0, D)],
                out_hbm.at[pl.ds(0, CHUNK)], ssem.at[b],
            ).wait()

    return k(idx2d, t_pad)


def kernel(variantxgene_ixs, table):
    idx2d = variantxgene_ixs.reshape(N_ROWS // CHUNK, CHUNK).astype(jnp.int32)
    t_pad = jnp.pad(table, ((0, 0), (0, D_PAD - D)))
    out = _sc_gather(idx2d, t_pad)
    return out.reshape(B, S, D)


# speedup vs baseline: 1.1730x; 1.0110x over previous
import functools

import jax
import jax.numpy as jnp
from jax import lax
from jax.experimental import pallas as pl
from jax.experimental.pallas import tpu as pltpu
from jax.experimental.pallas import tpu_sc as plsc

N_VARIANTXGENES = 1_000_000
B, S = 16384, 50
D = 64
N_ROWS = B * S
CHUNK = 128
NW = 32
CHUNKS_PER_W = N_ROWS // (CHUNK * NW)
NB = 4


def _sc_gather(idx2d, table):
    mesh = plsc.VectorSubcoreMesh(core_axis_name="c", subcore_axis_name="s")

    @functools.partial(
        pl.kernel,
        out_type=jax.ShapeDtypeStruct((N_ROWS, D), jnp.float32),
        mesh=mesh,
        scratch_types=[
            pltpu.VMEM((CHUNKS_PER_W, CHUNK), jnp.int32),
            pltpu.VMEM((NB, CHUNK, D), jnp.float32),
            pltpu.SemaphoreType.DMA((NB,)),
            pltpu.SemaphoreType.DMA((NB,)),
        ],
        compiler_params=pltpu.CompilerParams(use_tc_tiling_on_sc=False),
    )
    def k(idx_hbm, tbl, out_hbm, idx_v, gbuf, gsem, ssem):
        wid = lax.axis_index("s") * 2 + lax.axis_index("c")
        pltpu.sync_copy(idx_hbm.at[pl.ds(wid * CHUNKS_PER_W, CHUNKS_PER_W)], idx_v)

        def fire(b, j):
            pltpu.async_copy(tbl.at[idx_v.at[j]], gbuf.at[b], gsem.at[b])

        for b in range(NB):
            fire(b, b)

        def group(g, carry):
            for b in range(NB):
                j = g * NB + b
                pltpu.make_async_copy(
                    tbl.at[idx_v.at[0]], gbuf.at[b], gsem.at[b]
                ).wait()
                base = (wid * CHUNKS_PER_W + j) * CHUNK
                pltpu.async_copy(
                    gbuf.at[b], out_hbm.at[pl.ds(base, CHUNK)], ssem.at[b]
                )
                nxt = j + NB

                @pl.when(nxt < CHUNKS_PER_W)
                def _():
                    pltpu.make_async_copy(
                        gbuf.at[b], out_hbm.at[pl.ds(0, CHUNK)], ssem.at[b]
                    ).wait()
                    fire(b, nxt)

            return carry

        lax.fori_loop(0, CHUNKS_PER_W // NB, group, 0)

        for b in range(NB):
            pltpu.make_async_copy(
                gbuf.at[b], out_hbm.at[pl.ds(0, CHUNK)], ssem.at[b]
            ).wait()

    return k(idx2d, table)


def kernel(variantxgene_ixs, table):
    idx2d = variantxgene_ixs.reshape(N_ROWS // CHUNK, CHUNK).astype(jnp.int32)
    out = _sc_gather(idx2d, table)
    return out.reshape(B, S, D)
